# baseline (device time: 25836 ns/iter reference)
import jax
import jax.numpy as jnp
from jax import lax
from jax.experimental import pallas as pl
from jax.experimental.pallas import tpu as pltpu

N_DEV = 4
EPS = 1e-5
GLOBAL_C = 2048


def _stats_body(x_ref, out_ref, comm_ref, send_sems, recv_sems):
    my = lax.axis_index("i")
    left = (my - 1) % N_DEV
    right = (my + 1) % N_DEV

    barrier_sem = pltpu.get_barrier_semaphore()
    for nbr in [left, right]:
        pl.semaphore_signal(
            barrier_sem, inc=1,
            device_id=(nbr,), device_id_type=pl.DeviceIdType.MESH,
        )
    pl.semaphore_wait(barrier_sem, 2)

    x = x_ref[:]
    ssum = jnp.sum(x, axis=-1)
    ssq = jnp.sum(x * x, axis=-1)
    stats = jnp.concatenate([ssum, ssq], axis=0)
    comm_ref[0] = stats

    acc = stats
    for h in range(N_DEV - 1):
        rdma = pltpu.make_async_remote_copy(
            src_ref=comm_ref.at[h],
            dst_ref=comm_ref.at[h + 1],
            send_sem=send_sems.at[h],
            recv_sem=recv_sems.at[h],
            device_id=(right,),
            device_id_type=pl.DeviceIdType.MESH,
        )
        rdma.start()
        rdma.wait()
        acc = acc + comm_ref[h + 1]

    mean = acc[0:2] * (1.0 / GLOBAL_C)
    ex2 = acc[2:4] * (1.0 / GLOBAL_C)
    var = ex2 - mean * mean
    rstd = lax.rsqrt(var + EPS)
    out_ref[0] = rstd
    out_ref[1] = -mean * rstd


def _stats_call(x):
    b, s, c = x.shape
    return pl.pallas_call(
        _stats_body,
        out_shape=jax.ShapeDtypeStruct((2, b, s), jnp.float32),
        in_specs=[pl.BlockSpec(memory_space=pltpu.VMEM)],
        out_specs=pl.BlockSpec(memory_space=pltpu.VMEM),
        scratch_shapes=[
            pltpu.VMEM((N_DEV, 2 * b, s), jnp.float32),
            pltpu.SemaphoreType.DMA((N_DEV - 1,)),
            pltpu.SemaphoreType.DMA((N_DEV - 1,)),
        ],
        compiler_params=pltpu.CompilerParams(collective_id=0),
    )(x)


def _apply_body(x_ref, stats_ref, t_ref, ws_ref, wh_ref, out_ref):
    a = stats_ref[0]
    nb = stats_ref[1]
    scale = jnp.dot(t_ref[:], ws_ref[:], preferred_element_type=jnp.float32)
    shift = jnp.dot(t_ref[:], wh_ref[:], preferred_element_type=jnp.float32)
    x = x_ref[:]
    h = x * a[:, :, None] + nb[:, :, None]
    out = h * (1.0 + scale)[:, None, :] + shift[:, None, :]
    out_ref[:] = out.astype(out_ref.dtype)


def _apply_call(x, stats, t_emb, w_scale, w_shift):
    b, s, c = x.shape
    ch = 512
    grid = (s // ch,)
    return pl.pallas_call(
        _apply_body,
        grid=grid,
        out_shape=jax.ShapeDtypeStruct((b, s, c), jnp.bfloat16),
        in_specs=[
            pl.BlockSpec((b, ch, c), lambda i: (0, i, 0)),
            pl.BlockSpec((2, b, ch), lambda i: (0, 0, i)),
            pl.BlockSpec(t_emb.shape, lambda i: (0, 0)),
            pl.BlockSpec(w_scale.shape, lambda i: (0, 0)),
            pl.BlockSpec(w_shift.shape, lambda i: (0, 0)),
        ],
        out_specs=pl.BlockSpec((b, ch, c), lambda i: (0, i, 0)),
    )(x, stats, t_emb, w_scale, w_shift)


def kernel(x, t_emb, W_scale, W_shift):
    stats = _stats_call(x)
    return _apply_call(x, stats, t_emb, W_scale, W_shift)
